# Initial kernel scaffold; baseline (speedup 1.0000x reference)
import jax
import jax.numpy as jnp
from jax.experimental import pallas as pl


def kernel(node_states, edge_index, rel_ids, rel_emb, msg_W1, msg_b1, msg_W2, msg_b2, gate_W1, gate_b1, gate_W2, gate_b2, ln_g, ln_b, pool_W1, pool_b1, pool_W2, pool_b2):
    raise NotImplementedError("write your pallas kernel here")


# speedup vs baseline: 2.1561x; 2.1561x over previous
import jax
import jax.numpy as jnp
from jax import lax
from jax.experimental import pallas as pl
from jax.experimental.pallas import tpu as pltpu
from jax.experimental.pallas import tpu_sc as plsc

N = 10000
E = 320000
D = 128

NC = 2
NS = 16
NW = NC * NS

NB = 400
N_BLOCKS = N // NB


def _proj_body(h_ref, wsrc_ref, wgd_ref, rel8_ref, wrel_ref, brel_ref,
               psrc_ref, pgd_ref, reltab_ref):
    h = h_ref[...]
    psrc_ref[...] = jnp.dot(h, wsrc_ref[...], preferred_element_type=jnp.float32)
    pgd_ref[...] = jnp.dot(h, wgd_ref[...], preferred_element_type=jnp.float32)

    @pl.when(pl.program_id(0) == 0)
    def _():
        reltab_ref[...] = (
            jnp.dot(rel8_ref[...], wrel_ref[...], preferred_element_type=jnp.float32)
            + brel_ref[...])


def _proj_call(h, wsrc, wgd, rel8, wrel, brel):
    return pl.pallas_call(
        _proj_body,
        grid=(N_BLOCKS,),
        in_specs=[
            pl.BlockSpec((NB, D), lambda i: (i, 0)),
            pl.BlockSpec((D, 2 * D), lambda i: (0, 0)),
            pl.BlockSpec((D, D), lambda i: (0, 0)),
            pl.BlockSpec((8, D), lambda i: (0, 0)),
            pl.BlockSpec((D, 2 * D), lambda i: (0, 0)),
            pl.BlockSpec((1, 2 * D), lambda i: (0, 0)),
        ],
        out_specs=[
            pl.BlockSpec((NB, 2 * D), lambda i: (i, 0)),
            pl.BlockSpec((NB, D), lambda i: (i, 0)),
            pl.BlockSpec((8, 2 * D), lambda i: (0, 0)),
        ],
        out_shape=[
            jax.ShapeDtypeStruct((N, 2 * D), jnp.float32),
            jax.ShapeDtypeStruct((N, D), jnp.float32),
            jax.ShapeDtypeStruct((8, 2 * D), jnp.float32),
        ],
    )(h, wsrc, wgd, rel8, wrel, brel)


GC = 80
EPW = E // NW
GITERS = EPW // GC


def _gather_body(psrc_hbm, pgd_hbm, src_hbm, dst_hbm, gsrc_hbm, gdst_hbm,
                 sidx_v, didx_v, srows_v, drows_v, sem1, sem2):
    wid = lax.axis_index("s") * NC + lax.axis_index("c")

    def body(it, carry):
        base = pl.multiple_of(wid * EPW + it * GC, 8)
        pltpu.sync_copy(src_hbm.at[pl.ds(base, GC)], sidx_v)
        pltpu.sync_copy(dst_hbm.at[pl.ds(base, GC)], didx_v)
        cp1 = pltpu.async_copy(psrc_hbm.at[sidx_v], srows_v, sem1)
        cp2 = pltpu.async_copy(pgd_hbm.at[didx_v], drows_v, sem2)
        cp1.wait()
        cp2.wait()
        pltpu.sync_copy(srows_v, gsrc_hbm.at[pl.ds(base, GC)])
        pltpu.sync_copy(drows_v, gdst_hbm.at[pl.ds(base, GC)])
        return carry

    lax.fori_loop(0, GITERS, body, 0)


def _gather_call(psrc, pgd, src, dst):
    mesh = plsc.VectorSubcoreMesh(core_axis_name="c", subcore_axis_name="s")
    f = pl.kernel(
        _gather_body,
        out_type=[
            jax.ShapeDtypeStruct((E, 2 * D), jnp.float32),
            jax.ShapeDtypeStruct((E, D), jnp.float32),
        ],
        mesh=mesh,
        scratch_types=[
            pltpu.VMEM((GC,), jnp.int32),
            pltpu.VMEM((GC,), jnp.int32),
            pltpu.VMEM((GC, 2 * D), jnp.float32),
            pltpu.VMEM((GC, D), jnp.float32),
            pltpu.SemaphoreType.DMA,
            pltpu.SemaphoreType.DMA,
        ],
    )
    return f(psrc, pgd, src, dst)


EB = 512
E_BLOCKS = E // EB


def _edge_body(gsrc_ref, gdst_ref, rel_ref, reltab_ref, w2_ref, b2_ref,
               gw2_ref, gb2_ref, ge_ref):
    ids = rel_ref[0, 0, :]
    onehot = (ids[:, None] == lax.broadcasted_iota(jnp.int32, (EB, 8), 1)
              ).astype(jnp.float32)
    addend = jnp.dot(onehot, reltab_ref[...], preferred_element_type=jnp.float32)
    gsrc = gsrc_ref[...]
    u = jax.nn.gelu(gsrc[:, :D] + addend[:, :D])
    m = jnp.dot(u, w2_ref[...], preferred_element_type=jnp.float32) + b2_ref[...]
    v = jax.nn.gelu(gdst_ref[...] + gsrc[:, D:] + addend[:, D:])
    gsc = jnp.sum(v * gw2_ref[...], axis=-1, keepdims=True) + gb2_ref[...]
    ge_ref[...] = jax.nn.sigmoid(gsc) * m


def _edge_call(gsrc, gdst, rel3, reltab, w2, b2, gw2row, gb2):
    return pl.pallas_call(
        _edge_body,
        grid=(E_BLOCKS,),
        in_specs=[
            pl.BlockSpec((EB, 2 * D), lambda i: (i, 0)),
            pl.BlockSpec((EB, D), lambda i: (i, 0)),
            pl.BlockSpec((1, 1, EB), lambda i: (i, 0, 0)),
            pl.BlockSpec((8, 2 * D), lambda i: (0, 0)),
            pl.BlockSpec((D, D), lambda i: (0, 0)),
            pl.BlockSpec((1, D), lambda i: (0, 0)),
            pl.BlockSpec((1, D), lambda i: (0, 0)),
            pl.BlockSpec((1, 1), lambda i: (0, 0)),
        ],
        out_specs=pl.BlockSpec((EB, D), lambda i: (i, 0)),
        out_shape=jax.ShapeDtypeStruct((E, D), jnp.float32),
    )(gsrc, gdst, rel3, reltab, w2, b2, gw2row, gb2)


SCC = 80
EPT = E // (NC * NS)
SC_ITERS = EPT // SCC
NP = 10240
RPT = NP // NS


def _scatter_body(ge_hbm, dst_hbm, zeros_hbm, parts_hbm,
                  idx_v, rows_v, agg_sh):
    c = lax.axis_index("c")
    s = lax.axis_index("s")
    pltpu.sync_copy(zeros_hbm.at[pl.ds(s * RPT, RPT)],
                    agg_sh.at[pl.ds(s * RPT, RPT)])
    plsc.subcore_barrier()

    def body(it, carry):
        base = pl.multiple_of((c * NS + s) * EPT + it * SCC, 8)
        pltpu.sync_copy(dst_hbm.at[pl.ds(base, SCC)], idx_v)
        pltpu.sync_copy(ge_hbm.at[pl.ds(base, SCC)], rows_v)
        pltpu.sync_copy(rows_v, agg_sh.at[idx_v], add=True)
        return carry

    lax.fori_loop(0, SC_ITERS, body, 0)
    plsc.subcore_barrier()
    pltpu.sync_copy(agg_sh.at[pl.ds(s * RPT, RPT)],
                    parts_hbm.at[c, pl.ds(s * RPT, RPT)])


def _scatter_call(ge, dst, zeros_nd):
    mesh = plsc.VectorSubcoreMesh(core_axis_name="c", subcore_axis_name="s")
    f = pl.kernel(
        _scatter_body,
        out_type=jax.ShapeDtypeStruct((NC, NP, D), jnp.float32),
        mesh=mesh,
        scratch_types=[
            pltpu.VMEM((SCC,), jnp.int32),
            pltpu.VMEM((SCC, D), jnp.float32),
            pltpu.VMEM_SHARED((NP, D), jnp.float32),
        ],
    )
    return f(ge, dst, zeros_nd)


def _ln_body(h_ref, p0_ref, p1_ref, g_ref, b_ref, out_ref):
    x = h_ref[...] + p0_ref[0] + p1_ref[0]
    mu = jnp.mean(x, axis=-1, keepdims=True)
    xc = x - mu
    var = jnp.mean(xc * xc, axis=-1, keepdims=True)
    out_ref[...] = xc * lax.rsqrt(var + 1e-5) * g_ref[...] + b_ref[...]


def _ln_call(h, parts, g, b):
    return pl.pallas_call(
        _ln_body,
        grid=(N_BLOCKS,),
        in_specs=[
            pl.BlockSpec((NB, D), lambda i: (i, 0)),
            pl.BlockSpec((1, NB, D), lambda i: (0, i, 0)),
            pl.BlockSpec((1, NB, D), lambda i: (1, i, 0)),
            pl.BlockSpec((1, D), lambda i: (0, 0)),
            pl.BlockSpec((1, D), lambda i: (0, 0)),
        ],
        out_specs=pl.BlockSpec((NB, D), lambda i: (i, 0)),
        out_shape=jax.ShapeDtypeStruct((N, D), jnp.float32),
    )(h, parts, parts, g, b)


def _pool_body(h_ref, pw1_ref, pb1_ref, pw2_ref, pb2_ref, out_ref,
               sum_ref, max_ref):
    i = pl.program_id(0)

    @pl.when(i == 0)
    def _():
        sum_ref[...] = jnp.zeros_like(sum_ref)
        max_ref[...] = jnp.full_like(max_ref, -jnp.inf)

    blk = h_ref[...]
    sum_ref[...] += jnp.broadcast_to(jnp.sum(blk, axis=0, keepdims=True), (8, D))
    max_ref[...] = jnp.maximum(
        max_ref[...], jnp.broadcast_to(jnp.max(blk, axis=0, keepdims=True), (8, D)))

    @pl.when(i == N_BLOCKS - 1)
    def _():
        mean8 = sum_ref[...] * (1.0 / N)
        pin = jnp.concatenate([mean8, max_ref[...]], axis=-1)
        hdn = jax.nn.gelu(
            jnp.dot(pin, pw1_ref[...], preferred_element_type=jnp.float32)
            + pb1_ref[...])
        out_ref[...] = (
            jnp.dot(hdn, pw2_ref[...], preferred_element_type=jnp.float32)
            + pb2_ref[...])


def _pool_call(h, pw1, pb1, pw2, pb2):
    return pl.pallas_call(
        _pool_body,
        grid=(N_BLOCKS,),
        in_specs=[
            pl.BlockSpec((NB, D), lambda i: (i, 0)),
            pl.BlockSpec((2 * D, D), lambda i: (0, 0)),
            pl.BlockSpec((1, D), lambda i: (0, 0)),
            pl.BlockSpec((D, D), lambda i: (0, 0)),
            pl.BlockSpec((1, D), lambda i: (0, 0)),
        ],
        out_specs=pl.BlockSpec((8, D), lambda i: (0, 0)),
        out_shape=jax.ShapeDtypeStruct((8, D), jnp.float32),
        scratch_shapes=[
            pltpu.VMEM((8, D), jnp.float32),
            pltpu.VMEM((8, D), jnp.float32),
        ],
    )(h, pw1, pb1, pw2, pb2)


def kernel(node_states, edge_index, rel_ids, rel_emb,
           msg_W1, msg_b1, msg_W2, msg_b2,
           gate_W1, gate_b1, gate_W2, gate_b2,
           ln_g, ln_b, pool_W1, pool_b1, pool_W2, pool_b2):
    src = edge_index[0]
    dst = edge_index[1]
    rel3 = rel_ids.reshape(E // EB, 1, EB)
    rel8 = jnp.pad(rel_emb, ((0, 8 - rel_emb.shape[0]), (0, 0)))
    zeros_nd = jnp.zeros((NP, D), jnp.float32)

    h = node_states
    L = msg_W1.shape[0]
    for l in range(L):
        wsrc = jnp.concatenate([msg_W1[l][:D], gate_W1[l][D:2 * D]], axis=1)
        wgd = gate_W1[l][:D]
        wrel = jnp.concatenate([msg_W1[l][D:], gate_W1[l][2 * D:]], axis=1)
        brel = jnp.concatenate([msg_b1[l], gate_b1[l]])[None, :]
        psrc, pgd, reltab = _proj_call(h, wsrc, wgd, rel8, wrel, brel)
        gsrc, gdst = _gather_call(psrc, pgd, src, dst)
        ge = _edge_call(gsrc, gdst, rel3, reltab,
                        msg_W2[l], msg_b2[l][None, :],
                        gate_W2[l].T, gate_b2[l][None, :])
        parts = _scatter_call(ge, dst, zeros_nd)
        h = _ln_call(h, parts, ln_g[l][None, :], ln_b[l][None, :])

    pooled = _pool_call(h, pool_W1, pool_b1[None, :], pool_W2, pool_b2[None, :])
    return jnp.concatenate([h, pooled[:1]], axis=0)


# speedup vs baseline: 2.3881x; 1.1076x over previous
import jax
import jax.numpy as jnp
from jax import lax
from jax.experimental import pallas as pl
from jax.experimental.pallas import tpu as pltpu
from jax.experimental.pallas import tpu_sc as plsc

N = 10000
E = 320000
D = 128

NC = 2
NS = 16
NW = NC * NS

NB = 400
N_BLOCKS = N // NB


def _bf16_bits(x):
    b = lax.bitcast_convert_type(x, jnp.int32)
    b = b + jnp.int32(0x7FFF) + (lax.shift_right_logical(b, 16) & jnp.int32(1))
    return b & jnp.int32(-65536)


def _pack2(hi_f32, lo_f32):
    return _bf16_bits(hi_f32) | lax.shift_right_logical(_bf16_bits(lo_f32), 16)


def _unpack_hi(i32):
    return lax.bitcast_convert_type(i32 & jnp.int32(-65536), jnp.float32)


def _unpack_lo(i32):
    return lax.bitcast_convert_type(lax.shift_left(i32, 16), jnp.float32)


def _proj_body(h_ref, wsrc_ref, wgd_ref, rel8_ref, wrel_ref, brel_ref,
               psrc_ref, pgd_ref, reltab_ref):
    h = h_ref[...]
    ps = jnp.dot(h, wsrc_ref[...], preferred_element_type=jnp.float32)
    psrc_ref[...] = _pack2(ps[:, :D], ps[:, D:])
    pgd_ref[...] = jnp.dot(h, wgd_ref[...], preferred_element_type=jnp.float32)

    @pl.when(pl.program_id(0) == 0)
    def _():
        reltab_ref[...] = (
            jnp.dot(rel8_ref[...], wrel_ref[...], preferred_element_type=jnp.float32)
            + brel_ref[...])


def _proj_call(h, wsrc, wgd, rel8, wrel, brel):
    return pl.pallas_call(
        _proj_body,
        grid=(N_BLOCKS,),
        in_specs=[
            pl.BlockSpec((NB, D), lambda i: (i, 0)),
            pl.BlockSpec((D, 2 * D), lambda i: (0, 0)),
            pl.BlockSpec((D, D), lambda i: (0, 0)),
            pl.BlockSpec((8, D), lambda i: (0, 0)),
            pl.BlockSpec((D, 2 * D), lambda i: (0, 0)),
            pl.BlockSpec((1, 2 * D), lambda i: (0, 0)),
        ],
        out_specs=[
            pl.BlockSpec((NB, D), lambda i: (i, 0)),
            pl.BlockSpec((NB, D), lambda i: (i, 0)),
            pl.BlockSpec((8, 2 * D), lambda i: (0, 0)),
        ],
        out_shape=[
            jax.ShapeDtypeStruct((N, D), jnp.int32),
            jax.ShapeDtypeStruct((N, D), jnp.float32),
            jax.ShapeDtypeStruct((8, 2 * D), jnp.float32),
        ],
    )(h, wsrc, wgd, rel8, wrel, brel)


GC = 80
EPW = E // NW
GITERS = EPW // GC


def _gather_body(psrc_hbm, pgd_hbm, src_hbm, dst_hbm, gsrc_hbm, gdst_hbm,
                 sidx_v, didx_v, srows_v, drows_v, sem1, sem2):
    wid = lax.axis_index("s") * NC + lax.axis_index("c")

    def body(it, carry):
        base = pl.multiple_of(wid * EPW + it * GC, 8)
        pltpu.sync_copy(src_hbm.at[pl.ds(base, GC)], sidx_v)
        pltpu.sync_copy(dst_hbm.at[pl.ds(base, GC)], didx_v)
        cp1 = pltpu.async_copy(psrc_hbm.at[sidx_v], srows_v, sem1)
        cp2 = pltpu.async_copy(pgd_hbm.at[didx_v], drows_v, sem2)
        cp1.wait()
        cp2.wait()
        pltpu.sync_copy(srows_v, gsrc_hbm.at[pl.ds(base, GC)])
        pltpu.sync_copy(drows_v, gdst_hbm.at[pl.ds(base, GC)])
        return carry

    lax.fori_loop(0, GITERS, body, 0)


def _gather_call(psrc, pgd, src, dst):
    mesh = plsc.VectorSubcoreMesh(core_axis_name="c", subcore_axis_name="s")
    f = pl.kernel(
        _gather_body,
        out_type=[
            jax.ShapeDtypeStruct((E, D), jnp.int32),
            jax.ShapeDtypeStruct((E, D), jnp.float32),
        ],
        mesh=mesh,
        scratch_types=[
            pltpu.VMEM((GC,), jnp.int32),
            pltpu.VMEM((GC,), jnp.int32),
            pltpu.VMEM((GC, D), jnp.int32),
            pltpu.VMEM((GC, D), jnp.float32),
            pltpu.SemaphoreType.DMA,
            pltpu.SemaphoreType.DMA,
        ],
    )
    return f(psrc, pgd, src, dst)


EB = 512
E_BLOCKS = E // EB


def _edge_body(gsrc_ref, gdst_ref, rel_ref, reltab_ref, w2_ref, b2_ref,
               gw2_ref, gb2_ref, ge_ref):
    ids = rel_ref[0, 0, :]
    onehot = (ids[:, None] == lax.broadcasted_iota(jnp.int32, (EB, 8), 1)
              ).astype(jnp.float32)
    addend = jnp.dot(onehot, reltab_ref[...], preferred_element_type=jnp.float32)
    gi = gsrc_ref[...]
    u = jax.nn.gelu(_unpack_hi(gi) + addend[:, :D])
    m = jnp.dot(u.astype(jnp.bfloat16), w2_ref[...],
                preferred_element_type=jnp.float32) + b2_ref[...]
    v = jax.nn.gelu(gdst_ref[...] + _unpack_lo(gi) + addend[:, D:])
    gsc = jnp.sum(v * gw2_ref[...], axis=-1, keepdims=True) + gb2_ref[...]
    ge_ref[...] = jax.nn.sigmoid(gsc) * m


def _edge_call(gsrc, gdst, rel3, reltab, w2, b2, gw2row, gb2):
    return pl.pallas_call(
        _edge_body,
        grid=(E_BLOCKS,),
        in_specs=[
            pl.BlockSpec((EB, D), lambda i: (i, 0)),
            pl.BlockSpec((EB, D), lambda i: (i, 0)),
            pl.BlockSpec((1, 1, EB), lambda i: (i, 0, 0)),
            pl.BlockSpec((8, 2 * D), lambda i: (0, 0)),
            pl.BlockSpec((D, D), lambda i: (0, 0)),
            pl.BlockSpec((1, D), lambda i: (0, 0)),
            pl.BlockSpec((1, D), lambda i: (0, 0)),
            pl.BlockSpec((1, 1), lambda i: (0, 0)),
        ],
        out_specs=pl.BlockSpec((EB, D), lambda i: (i, 0)),
        out_shape=jax.ShapeDtypeStruct((E, D), jnp.float32),
    )(gsrc, gdst, rel3, reltab, w2, b2, gw2row, gb2)


SCC = 80
EPT = E // (NC * NS)
SC_ITERS = EPT // SCC
NP = 10240
RPT = NP // NS


def _scatter_body(ge_hbm, dst_hbm, zeros_hbm, parts_hbm,
                  idx_v, rows_v, agg_sh):
    c = lax.axis_index("c")
    s = lax.axis_index("s")
    pltpu.sync_copy(zeros_hbm.at[pl.ds(s * RPT, RPT)],
                    agg_sh.at[pl.ds(s * RPT, RPT)])
    plsc.subcore_barrier()

    def body(it, carry):
        base = pl.multiple_of((c * NS + s) * EPT + it * SCC, 8)
        pltpu.sync_copy(dst_hbm.at[pl.ds(base, SCC)], idx_v)
        pltpu.sync_copy(ge_hbm.at[pl.ds(base, SCC)], rows_v)
        pltpu.sync_copy(rows_v, agg_sh.at[idx_v], add=True)
        return carry

    lax.fori_loop(0, SC_ITERS, body, 0)
    plsc.subcore_barrier()
    pltpu.sync_copy(agg_sh.at[pl.ds(s * RPT, RPT)],
                    parts_hbm.at[c, pl.ds(s * RPT, RPT)])


def _scatter_call(ge, dst, zeros_nd):
    mesh = plsc.VectorSubcoreMesh(core_axis_name="c", subcore_axis_name="s")
    f = pl.kernel(
        _scatter_body,
        out_type=jax.ShapeDtypeStruct((NC, NP, D), jnp.float32),
        mesh=mesh,
        scratch_types=[
            pltpu.VMEM((SCC,), jnp.int32),
            pltpu.VMEM((SCC, D), jnp.float32),
            pltpu.VMEM_SHARED((NP, D), jnp.float32),
        ],
    )
    return f(ge, dst, zeros_nd)


def _ln_body(h_ref, p0_ref, p1_ref, g_ref, b_ref, out_ref):
    x = h_ref[...] + p0_ref[0] + p1_ref[0]
    mu = jnp.mean(x, axis=-1, keepdims=True)
    xc = x - mu
    var = jnp.mean(xc * xc, axis=-1, keepdims=True)
    out_ref[...] = xc * lax.rsqrt(var + 1e-5) * g_ref[...] + b_ref[...]


def _ln_call(h, parts, g, b):
    return pl.pallas_call(
        _ln_body,
        grid=(N_BLOCKS,),
        in_specs=[
            pl.BlockSpec((NB, D), lambda i: (i, 0)),
            pl.BlockSpec((1, NB, D), lambda i: (0, i, 0)),
            pl.BlockSpec((1, NB, D), lambda i: (1, i, 0)),
            pl.BlockSpec((1, D), lambda i: (0, 0)),
            pl.BlockSpec((1, D), lambda i: (0, 0)),
        ],
        out_specs=pl.BlockSpec((NB, D), lambda i: (i, 0)),
        out_shape=jax.ShapeDtypeStruct((N, D), jnp.float32),
    )(h, parts, parts, g, b)


def _pool_body(h_ref, pw1_ref, pb1_ref, pw2_ref, pb2_ref, out_ref,
               sum_ref, max_ref):
    i = pl.program_id(0)

    @pl.when(i == 0)
    def _():
        sum_ref[...] = jnp.zeros_like(sum_ref)
        max_ref[...] = jnp.full_like(max_ref, -jnp.inf)

    blk = h_ref[...]
    sum_ref[...] += jnp.broadcast_to(jnp.sum(blk, axis=0, keepdims=True), (8, D))
    max_ref[...] = jnp.maximum(
        max_ref[...], jnp.broadcast_to(jnp.max(blk, axis=0, keepdims=True), (8, D)))

    @pl.when(i == N_BLOCKS - 1)
    def _():
        mean8 = sum_ref[...] * (1.0 / N)
        pin = jnp.concatenate([mean8, max_ref[...]], axis=-1)
        hdn = jax.nn.gelu(
            jnp.dot(pin, pw1_ref[...], preferred_element_type=jnp.float32)
            + pb1_ref[...])
        out_ref[...] = (
            jnp.dot(hdn, pw2_ref[...], preferred_element_type=jnp.float32)
            + pb2_ref[...])


def _pool_call(h, pw1, pb1, pw2, pb2):
    return pl.pallas_call(
        _pool_body,
        grid=(N_BLOCKS,),
        in_specs=[
            pl.BlockSpec((NB, D), lambda i: (i, 0)),
            pl.BlockSpec((2 * D, D), lambda i: (0, 0)),
            pl.BlockSpec((1, D), lambda i: (0, 0)),
            pl.BlockSpec((D, D), lambda i: (0, 0)),
            pl.BlockSpec((1, D), lambda i: (0, 0)),
        ],
        out_specs=pl.BlockSpec((8, D), lambda i: (0, 0)),
        out_shape=jax.ShapeDtypeStruct((8, D), jnp.float32),
        scratch_shapes=[
            pltpu.VMEM((8, D), jnp.float32),
            pltpu.VMEM((8, D), jnp.float32),
        ],
    )(h, pw1, pb1, pw2, pb2)


def kernel(node_states, edge_index, rel_ids, rel_emb,
           msg_W1, msg_b1, msg_W2, msg_b2,
           gate_W1, gate_b1, gate_W2, gate_b2,
           ln_g, ln_b, pool_W1, pool_b1, pool_W2, pool_b2):
    src = edge_index[0]
    dst = edge_index[1]
    rel3 = rel_ids.reshape(E // EB, 1, EB)
    rel8 = jnp.pad(rel_emb, ((0, 8 - rel_emb.shape[0]), (0, 0)))
    zeros_nd = jnp.zeros((NP, D), jnp.float32)

    h = node_states
    L = msg_W1.shape[0]
    for l in range(L):
        wsrc = jnp.concatenate([msg_W1[l][:D], gate_W1[l][D:2 * D]], axis=1)
        wgd = gate_W1[l][:D]
        wrel = jnp.concatenate([msg_W1[l][D:], gate_W1[l][2 * D:]], axis=1)
        brel = jnp.concatenate([msg_b1[l], gate_b1[l]])[None, :]
        psrc, pgd, reltab = _proj_call(h, wsrc, wgd, rel8, wrel, brel)
        gsrc, gdst = _gather_call(psrc, pgd, src, dst)
        ge = _edge_call(gsrc, gdst, rel3, reltab,
                        msg_W2[l].astype(jnp.bfloat16), msg_b2[l][None, :],
                        gate_W2[l].T, gate_b2[l][None, :])
        parts = _scatter_call(ge, dst, zeros_nd)
        h = _ln_call(h, parts, ln_g[l][None, :], ln_b[l][None, :])

    pooled = _pool_call(h, pool_W1, pool_b1[None, :], pool_W2, pool_b2[None, :])
    return jnp.concatenate([h, pooled[:1]], axis=0)
